# baseline (speedup 1.0000x reference)
import jax
import jax.numpy as jnp
from jax import lax
from jax.experimental import pallas as pl
from jax.experimental.pallas import tpu as pltpu
from jax.experimental.pallas import tpu_sc as plsc

N = 10000
E = 160000
D_IN = 256
HID = 64
HEADS = 8
D_MID = 512
D_OUT = 256

NC = 2
NS = 16
NW = NC * NS
BA = 32
BB = 64
EPW = 5120
EP = NW * EPW
NBB = EPW // BB
NB0 = 104
NB1 = 56
NP = 10240
RP = NP // NS

BN = 400
GRID = N // BN

_f32 = jnp.float32
_i32 = jnp.int32


def _tc1(x, Wcat, bcat):
    def body(x_ref, w_ref, b_ref, q_ref, k_ref, s_ref, *v_refs):
        y = jnp.dot(x_ref[...], w_ref[...], preferred_element_type=_f32)
        y = y + b_ref[...]
        q_ref[...] = y[:, 0:D_MID].astype(jnp.bfloat16)
        k_ref[...] = y[:, D_MID:2 * D_MID].astype(jnp.bfloat16)
        s_ref[...] = y[:, 2 * D_MID:3 * D_MID]
        for i, vr in enumerate(v_refs):
            vr[...] = y[:, 3 * D_MID + i * 64:3 * D_MID + (i + 1) * 64]

    return pl.pallas_call(
        body,
        grid=(GRID,),
        in_specs=[
            pl.BlockSpec((BN, D_IN), lambda i: (i, 0)),
            pl.BlockSpec((D_IN, 4 * D_MID), lambda i: (0, 0)),
            pl.BlockSpec((1, 4 * D_MID), lambda i: (0, 0)),
        ],
        out_specs=[pl.BlockSpec((BN, D_MID), lambda i: (i, 0))] * 3
        + [pl.BlockSpec((BN, 64), lambda i: (i, 0))] * 8,
        out_shape=[jax.ShapeDtypeStruct((N, D_MID), jnp.bfloat16)] * 2
        + [jax.ShapeDtypeStruct((N, D_MID), _f32)]
        + [jax.ShapeDtypeStruct((N, 64), _f32)] * 8,
    )(x, Wcat, bcat)


def _tc2(outp, denp, s1, Wcat, bcat):
    def body(op_ref, dp_ref, s_ref, w_ref, b_ref, q_ref, k_ref, s2_ref, *v_refs):
        d = dp_ref[0] + dp_ref[1]
        recd = 1.0 / (d + 1e-16)
        y = jnp.broadcast_to(b_ref[...], (BN, 4 * D_OUT))
        for g in range(HEADS):
            hg = (op_ref[0, g] + op_ref[1, g]) * recd[:, g:g + 1]
            hg = jnp.maximum(hg + s_ref[:, g * HID:(g + 1) * HID], 0.0)
            y = y + jnp.dot(hg, w_ref[pl.ds(g * HID, HID), :],
                            preferred_element_type=_f32)
        q_ref[...] = y[:, 0:D_OUT].astype(jnp.bfloat16)
        k_ref[...] = y[:, D_OUT:2 * D_OUT].astype(jnp.bfloat16)
        s2_ref[...] = y[:, 2 * D_OUT:3 * D_OUT]
        for i, vr in enumerate(v_refs):
            vr[...] = y[:, 3 * D_OUT + i * 64:3 * D_OUT + (i + 1) * 64]

    return pl.pallas_call(
        body,
        grid=(GRID,),
        in_specs=[
            pl.BlockSpec((NC, HEADS, BN, HID), lambda i: (0, 0, i, 0)),
            pl.BlockSpec((NC, BN, 16), lambda i: (0, i, 0)),
            pl.BlockSpec((BN, D_MID), lambda i: (i, 0)),
            pl.BlockSpec((D_MID, 4 * D_OUT), lambda i: (0, 0)),
            pl.BlockSpec((1, 4 * D_OUT), lambda i: (0, 0)),
        ],
        out_specs=[pl.BlockSpec((BN, D_OUT), lambda i: (i, 0))] * 3
        + [pl.BlockSpec((BN, 64), lambda i: (i, 0))] * 4,
        out_shape=[jax.ShapeDtypeStruct((N, D_OUT), jnp.bfloat16)] * 2
        + [jax.ShapeDtypeStruct((N, D_OUT), _f32)]
        + [jax.ShapeDtypeStruct((N, 64), _f32)] * 4,
    )(outp, denp, s1, Wcat, bcat)


def _tc3(outp, denp, s2):
    def body(op_ref, dp_ref, s_ref, o_ref):
        d = dp_ref[0] + dp_ref[1]
        rec = 1.0 / (d[:, 0:1] + 1e-16)
        cols = []
        for g in range(4):
            pg = op_ref[0, g] + op_ref[1, g]
            cols.append(jnp.maximum(pg * rec + s_ref[:, g * 64:(g + 1) * 64],
                                    0.0))
        o_ref[...] = jnp.concatenate(cols, axis=1)

    return pl.pallas_call(
        body,
        grid=(GRID,),
        in_specs=[
            pl.BlockSpec((NC, 4, BN, 64), lambda i: (0, 0, i, 0)),
            pl.BlockSpec((NC, BN, 16), lambda i: (0, i, 0)),
            pl.BlockSpec((BN, D_OUT), lambda i: (i, 0)),
        ],
        out_specs=pl.BlockSpec((BN, D_OUT), lambda i: (i, 0)),
        out_shape=jax.ShapeDtypeStruct((N, D_OUT), _f32),
    )(outp, denp, s2)


def _make_sc_layer(D, heads, ngrp, scale):
    ch = D // heads
    mesh = plsc.VectorSubcoreMesh(core_axis_name="c", subcore_axis_name="s")
    QPB = BB // BA

    def body(*refs):
        q_hbm, k_hbm = refs[0], refs[1]
        v_hbms = refs[2:2 + ngrp]
        (srcb_hbm, dstb_hbm, z16, z64,
         denom_hbm, out_hbm, a_hbm,
         srcb, dstb, Qa, Ka, Qb, Kb, ap0, ap1, lgt,
         Vv0, Vv1, Vv2, Vv3, mg0, mg1, av0, av1, av2, av3,
         denom_s, out_s,
         sq0, sk0, sq1, sk1, ssc0, ssc1, sst0, sst1,
         sv0, sv1, sv2, sv3, sa0, sa1, sa2, sa3, sm0, sm1) = refs[2 + ngrp:]

        Qbuf, Kbuf = (Qa, Qb), (Ka, Kb)
        apb = (ap0, ap1)
        Vb, mgb, avb = (Vv0, Vv1, Vv2, Vv3), (mg0, mg1), (av0, av1, av2, av3)
        sqb, skb = (sq0, sq1), (sk0, sk1)
        sscb, sstb = (ssc0, ssc1), (sst0, sst1)
        svb, sab, smb = (sv0, sv1, sv2, sv3), (sa0, sa1, sa2, sa3), (sm0, sm1)

        c = lax.axis_index("c")
        sid = lax.axis_index("s")
        row0 = sid * RP
        cnt = jnp.where(c == 0, NB0, NB1)
        blk0b = jnp.where(c == 0, sid * NB0, NS * NB0 + sid * NB1)
        iota16 = lax.iota(_i32, 16)

        pltpu.sync_copy(srcb_hbm.at[pl.ds(blk0b, NB0)], srcb)
        pltpu.sync_copy(dstb_hbm.at[pl.ds(blk0b, NB0)], dstb)

        pltpu.sync_copy(z16.at[pl.ds(row0, RP)], denom_s.at[pl.ds(row0, RP)])
        plsc.subcore_barrier()

        pltpu.async_copy(q_hbm.at[dstb.at[0, pl.ds(0, BA)]], Qbuf[0], sqb[0])
        pltpu.async_copy(k_hbm.at[srcb.at[0, pl.ds(0, BA)]], Kbuf[0], skb[0])

        def _pair_a(bp, carry):
            for hb in range(2):
                b = bp * 2 + hb
                pb = hb

                @pl.when(bp > 0)
                def _wait_prev():
                    pltpu.make_async_copy(
                        apb[pb], denom_s.at[dstb.at[0]], sscb[pb]).wait()
                    pltpu.make_async_copy(
                        apb[pb], a_hbm.at[pl.ds(0, BB)], sstb[pb]).wait()

                for hf in range(QPB):
                    gb = hf % 2
                    nb_ = (hf + 1) % 2
                    pltpu.make_async_copy(
                        q_hbm.at[dstb.at[0, pl.ds(0, BA)]], Qbuf[gb],
                        sqb[gb]).wait()
                    pltpu.make_async_copy(
                        k_hbm.at[srcb.at[0, pl.ds(0, BA)]], Kbuf[gb],
                        skb[gb]).wait()
                    if hf < QPB - 1:
                        nxb, nxo = b, (hf + 1) * BA
                    else:
                        nxb, nxo = jnp.minimum(b + 1, cnt - 1), 0
                    pltpu.async_copy(q_hbm.at[dstb.at[nxb, pl.ds(nxo, BA)]],
                                     Qbuf[nb_], sqb[nb_])
                    pltpu.async_copy(k_hbm.at[srcb.at[nxb, pl.ds(nxo, BA)]],
                                     Kbuf[nb_], skb[nb_])

                    Qg, Kg = Qbuf[gb], Kbuf[gb]

                    def _dotrow(e, ecarry, Qg=Qg, Kg=Kg, hf=hf):
                        vec = jnp.zeros((16,), _f32)
                        for h in range(heads - 1, -1, -1):
                            c0 = h * ch
                            p = jnp.zeros((16,), _f32)
                            for u in range(ch // 32):
                                qv = Qg[e, pl.ds(c0 + u * 32, 32)]
                                kv = Kg[e, pl.ds(c0 + u * 32, 32)]
                                qa, qb = plsc.unpack(
                                    qv, format=plsc.PackFormat.INTERLEAVED,
                                    preferred_element_type=_f32)
                                ka, kb = plsc.unpack(
                                    kv, format=plsc.PackFormat.INTERLEAVED,
                                    preferred_element_type=_f32)
                                p = p + qa * ka + qb * kb
                            vec = jnp.where(iota16 == h, jnp.sum(p), vec)
                        lgt[hf * BA + e, :] = vec
                        return ecarry
                    lax.fori_loop(0, BA, _dotrow, 0)

                eidb = (blk0b + b) * BB
                lmask = lax.iota(_i32, 16) < heads

                def _expb(r, ecarry):
                    ok = jnp.logical_and(lmask, (eidb + r) < E)
                    apb[pb][r, :] = jnp.where(ok, jnp.exp(lgt[r, :] * scale),
                                              0.0)
                    return ecarry
                lax.fori_loop(0, BB, _expb, 0, unroll=4)

                pltpu.async_copy(apb[pb], denom_s.at[dstb.at[b]], sscb[pb],
                                 add=True)
                pltpu.async_copy(apb[pb],
                                 a_hbm.at[pl.ds((blk0b + b) * BB, BB)],
                                 sstb[pb])
            return carry
        lax.fori_loop(0, cnt // 2, _pair_a, 0)

        pltpu.make_async_copy(q_hbm.at[dstb.at[0, pl.ds(0, BA)]], Qbuf[0],
                              sqb[0]).wait()
        pltpu.make_async_copy(k_hbm.at[srcb.at[0, pl.ds(0, BA)]], Kbuf[0],
                              skb[0]).wait()
        for pb in range(2):
            pltpu.make_async_copy(apb[pb], denom_s.at[dstb.at[0]],
                                  sscb[pb]).wait()
            pltpu.make_async_copy(apb[pb], a_hbm.at[pl.ds(0, BB)],
                                  sstb[pb]).wait()

        plsc.subcore_barrier()
        pltpu.sync_copy(denom_s.at[pl.ds(row0, RP)],
                        denom_hbm.at[c, pl.ds(row0, RP)])

        for g in range(ngrp):
            pltpu.sync_copy(z64.at[pl.ds(row0, RP)], out_s.at[pl.ds(row0, RP)])
            plsc.subcore_barrier()

            for pr in range(3):
                pltpu.async_copy(v_hbms[g].at[srcb.at[pr]], Vb[pr], svb[pr])
                pltpu.async_copy(a_hbm.at[pl.ds((blk0b + pr) * BB, BB)],
                                 avb[pr], sab[pr])

            def _quad_b(bp, carry, g=g):
                for qb in range(4):
                    b = bp * 4 + qb
                    u_ = qb
                    m_ = qb % 2

                    if qb < 2:
                        @pl.when(bp > 0)
                        def _wait_msg():
                            pltpu.make_async_copy(
                                mgb[m_], out_s.at[dstb.at[0]], smb[m_]).wait()
                    else:
                        pltpu.make_async_copy(
                            mgb[m_], out_s.at[dstb.at[0]], smb[m_]).wait()

                    pltpu.make_async_copy(v_hbms[g].at[srcb.at[0]], Vb[u_],
                                          svb[u_]).wait()
                    pltpu.make_async_copy(a_hbm.at[pl.ds(0, BB)], avb[u_],
                                          sab[u_]).wait()
                    nxb = jnp.minimum(b + 3, cnt - 1)
                    nu = (qb + 3) % 4
                    pltpu.async_copy(v_hbms[g].at[srcb.at[nxb]], Vb[nu],
                                     svb[nu])
                    pltpu.async_copy(a_hbm.at[pl.ds((blk0b + nxb) * BB, BB)],
                                     avb[nu], sab[nu])

                    Vg, Mg, Ag = Vb[u_], mgb[m_], avb[u_]

                    def _edge(e, ecarry, Vg=Vg, Mg=Mg, Ag=Ag, g=g):
                        arow = Ag[e, :]
                        for u in range(4):
                            h = (g * 64 + u * 16) // ch
                            Mg[e, pl.ds(u * 16, 16)] = (
                                Vg[e, pl.ds(u * 16, 16)] * arow[h])
                        return ecarry
                    lax.fori_loop(0, BB, _edge, 0, unroll=2)
                    pltpu.async_copy(Mg, out_s.at[dstb.at[b]], smb[m_],
                                     add=True)
                return carry
            lax.fori_loop(0, cnt // 4, _quad_b, 0)

            for pr in range(3):
                pltpu.make_async_copy(v_hbms[g].at[srcb.at[0]], Vb[pr],
                                      svb[pr]).wait()
                pltpu.make_async_copy(a_hbm.at[pl.ds(0, BB)], avb[pr],
                                      sab[pr]).wait()
            for m_ in range(2):
                pltpu.make_async_copy(mgb[m_], out_s.at[dstb.at[0]],
                                      smb[m_]).wait()

            plsc.subcore_barrier()
            pltpu.sync_copy(out_s.at[pl.ds(row0, RP)],
                            out_hbm.at[c, g, pl.ds(row0, RP)])

    return pl.kernel(
        body,
        compiler_params=pltpu.CompilerParams(use_tc_tiling_on_sc=False,
                                             needs_layout_passes=False),
        out_type=[
            jax.ShapeDtypeStruct((NC, NP, 16), _f32),
            jax.ShapeDtypeStruct((NC, ngrp, NP, 64), _f32),
            jax.ShapeDtypeStruct((EP, 16), _f32),
        ],
        mesh=mesh,
        scratch_types=[
            pltpu.VMEM((NB0, BB), _i32),
            pltpu.VMEM((NB0, BB), _i32),
            pltpu.VMEM((BA, D), jnp.bfloat16),
            pltpu.VMEM((BA, D), jnp.bfloat16),
            pltpu.VMEM((BA, D), jnp.bfloat16),
            pltpu.VMEM((BA, D), jnp.bfloat16),
            pltpu.VMEM((BB, 16), _f32),
            pltpu.VMEM((BB, 16), _f32),
            pltpu.VMEM((BB, 16), _f32),
            pltpu.VMEM((BB, 64), _f32),
            pltpu.VMEM((BB, 64), _f32),
            pltpu.VMEM((BB, 64), _f32),
            pltpu.VMEM((BB, 64), _f32),
            pltpu.VMEM((BB, 64), _f32),
            pltpu.VMEM((BB, 64), _f32),
            pltpu.VMEM((BB, 16), _f32),
            pltpu.VMEM((BB, 16), _f32),
            pltpu.VMEM((BB, 16), _f32),
            pltpu.VMEM((BB, 16), _f32),
            pltpu.VMEM_SHARED((NP, 16), _f32),
            pltpu.VMEM_SHARED((NP, 64), _f32),
        ] + [pltpu.SemaphoreType.DMA] * 18,
    )


def kernel(x, edge_index, Wq1, bq1, Wk1, bk1, Wv1, bv1, Ws1, bs1,
           Wq2, bq2, Wk2, bk2, Wv2, bv2, Ws2, bs2):
    Wcat1 = jnp.concatenate([Wq1, Wk1, Ws1, Wv1], axis=1)
    bcat1 = jnp.concatenate([bq1, bk1, bs1, bv1]).reshape(1, -1)
    Wcat2 = jnp.concatenate([Wq2, Wk2, Ws2, Wv2], axis=1)
    bcat2 = jnp.concatenate([bq2, bk2, bs2, bv2]).reshape(1, -1)

    pad = jnp.zeros((EP - E,), _i32)
    srcb2d = jnp.concatenate([edge_index[0], pad]).reshape(NW * NBB, BB)
    dstb2d = jnp.concatenate([edge_index[1], pad]).reshape(NW * NBB, BB)
    z16 = jnp.zeros((NP, 16), _f32)
    z64 = jnp.zeros((NP, 64), _f32)

    tc1_out = _tc1(x, Wcat1, bcat1)
    q1, k1, s1 = tc1_out[0], tc1_out[1], tc1_out[2]
    v1g = tc1_out[3:]
    sc1 = _make_sc_layer(D_MID, HEADS, 8, 1.0 / 8.0)
    denp1, outp1, _ = sc1(q1, k1, *v1g, srcb2d, dstb2d, z16, z64)

    tc2_out = _tc2(outp1, denp1, s1, Wcat2, bcat2)
    q2, k2, s2 = tc2_out[0], tc2_out[1], tc2_out[2]
    v2g = tc2_out[3:]
    sc2 = _make_sc_layer(D_OUT, 1, 4, 1.0 / 16.0)
    denp2, outp2, _ = sc2(q2, k2, *v2g, srcb2d, dstb2d, z16, z64)

    return _tc3(outp2, denp2, s2)


# speedup vs baseline: 1.0468x; 1.0468x over previous
import jax
import jax.numpy as jnp
from jax import lax
from jax.experimental import pallas as pl
from jax.experimental.pallas import tpu as pltpu
from jax.experimental.pallas import tpu_sc as plsc

N = 10000
E = 160000
D_IN = 256
HID = 64
HEADS = 8
D_MID = 512
D_OUT = 256

NC = 2
NS = 16
NW = NC * NS
BA = 32
BB = 64
EPW = 5120
EP = NW * EPW
NBB = EPW // BB
NB0 = 96
NB1 = 64
NP = 10240
RP = NP // NS

BN = 400
GRID = N // BN

_f32 = jnp.float32
_i32 = jnp.int32


def _tc1(x, Wcat, bcat):
    def body(x_ref, w_ref, b_ref, q_ref, k_ref, s_ref, *v_refs):
        y = jnp.dot(x_ref[...], w_ref[...], preferred_element_type=_f32)
        y = y + b_ref[...]
        q_ref[...] = y[:, 0:D_MID].astype(jnp.bfloat16)
        k_ref[...] = y[:, D_MID:2 * D_MID].astype(jnp.bfloat16)
        s_ref[...] = y[:, 2 * D_MID:3 * D_MID]
        for i, vr in enumerate(v_refs):
            vr[...] = y[:, 3 * D_MID + i * 64:3 * D_MID + (i + 1) * 64]

    return pl.pallas_call(
        body,
        grid=(GRID,),
        in_specs=[
            pl.BlockSpec((BN, D_IN), lambda i: (i, 0)),
            pl.BlockSpec((D_IN, 4 * D_MID), lambda i: (0, 0)),
            pl.BlockSpec((1, 4 * D_MID), lambda i: (0, 0)),
        ],
        out_specs=[pl.BlockSpec((BN, D_MID), lambda i: (i, 0))] * 3
        + [pl.BlockSpec((BN, 64), lambda i: (i, 0))] * 8,
        out_shape=[jax.ShapeDtypeStruct((N, D_MID), jnp.bfloat16)] * 2
        + [jax.ShapeDtypeStruct((N, D_MID), _f32)]
        + [jax.ShapeDtypeStruct((N, 64), _f32)] * 8,
    )(x, Wcat, bcat)


def _tc2(outp, denp, s1, Wcat, bcat):
    def body(op_ref, dp_ref, s_ref, w_ref, b_ref, q_ref, k_ref, s2_ref, *v_refs):
        d = dp_ref[0] + dp_ref[1]
        recd = 1.0 / (d + 1e-16)
        y = jnp.broadcast_to(b_ref[...], (BN, 4 * D_OUT))
        for g in range(HEADS):
            hg = (op_ref[0, g] + op_ref[1, g]) * recd[:, g:g + 1]
            hg = jnp.maximum(hg + s_ref[:, g * HID:(g + 1) * HID], 0.0)
            y = y + jnp.dot(hg, w_ref[pl.ds(g * HID, HID), :],
                            preferred_element_type=_f32)
        q_ref[...] = y[:, 0:D_OUT].astype(jnp.bfloat16)
        k_ref[...] = y[:, D_OUT:2 * D_OUT].astype(jnp.bfloat16)
        s2_ref[...] = y[:, 2 * D_OUT:3 * D_OUT]
        for i, vr in enumerate(v_refs):
            vr[...] = y[:, 3 * D_OUT + i * 64:3 * D_OUT + (i + 1) * 64]

    return pl.pallas_call(
        body,
        grid=(GRID,),
        in_specs=[
            pl.BlockSpec((NC, HEADS, BN, HID), lambda i: (0, 0, i, 0)),
            pl.BlockSpec((NC, BN, 16), lambda i: (0, i, 0)),
            pl.BlockSpec((BN, D_MID), lambda i: (i, 0)),
            pl.BlockSpec((D_MID, 4 * D_OUT), lambda i: (0, 0)),
            pl.BlockSpec((1, 4 * D_OUT), lambda i: (0, 0)),
        ],
        out_specs=[pl.BlockSpec((BN, D_OUT), lambda i: (i, 0))] * 3
        + [pl.BlockSpec((BN, 64), lambda i: (i, 0))] * 4,
        out_shape=[jax.ShapeDtypeStruct((N, D_OUT), jnp.bfloat16)] * 2
        + [jax.ShapeDtypeStruct((N, D_OUT), _f32)]
        + [jax.ShapeDtypeStruct((N, 64), _f32)] * 4,
    )(outp, denp, s1, Wcat, bcat)


def _tc3(outp, denp, s2):
    def body(op_ref, dp_ref, s_ref, o_ref):
        d = dp_ref[0] + dp_ref[1]
        rec = 1.0 / (d[:, 0:1] + 1e-16)
        cols = []
        for g in range(4):
            pg = op_ref[0, g] + op_ref[1, g]
            cols.append(jnp.maximum(pg * rec + s_ref[:, g * 64:(g + 1) * 64],
                                    0.0))
        o_ref[...] = jnp.concatenate(cols, axis=1)

    return pl.pallas_call(
        body,
        grid=(GRID,),
        in_specs=[
            pl.BlockSpec((NC, 4, BN, 64), lambda i: (0, 0, i, 0)),
            pl.BlockSpec((NC, BN, 16), lambda i: (0, i, 0)),
            pl.BlockSpec((BN, D_OUT), lambda i: (i, 0)),
        ],
        out_specs=pl.BlockSpec((BN, D_OUT), lambda i: (i, 0)),
        out_shape=jax.ShapeDtypeStruct((N, D_OUT), _f32),
    )(outp, denp, s2)


def _make_sc_layer(D, heads, ngrp, scale):
    ch = D // heads
    mesh = plsc.VectorSubcoreMesh(core_axis_name="c", subcore_axis_name="s")
    QPB = BB // BA

    def body(*refs):
        q_hbm, k_hbm = refs[0], refs[1]
        v_hbms = refs[2:2 + ngrp]
        (srcb_hbm, dstb_hbm, z16, z64,
         denom_hbm, out_hbm, a_hbm,
         srcb, dstb, Qa, Ka, Qb, Kb, ap0, ap1, lgt,
         Vv0, Vv1, Vv2, Vv3, mg0, mg1, av0, av1, av2, av3,
         denom_s, out_s,
         sq0, sk0, sq1, sk1, ssc0, ssc1, sst0, sst1,
         sv0, sv1, sv2, sv3, sa0, sa1, sa2, sa3, sm0, sm1) = refs[2 + ngrp:]

        Qbuf, Kbuf = (Qa, Qb), (Ka, Kb)
        apb = (ap0, ap1)
        Vb, mgb, avb = (Vv0, Vv1, Vv2, Vv3), (mg0, mg1), (av0, av1, av2, av3)
        sqb, skb = (sq0, sq1), (sk0, sk1)
        sscb, sstb = (ssc0, ssc1), (sst0, sst1)
        svb, sab, smb = (sv0, sv1, sv2, sv3), (sa0, sa1, sa2, sa3), (sm0, sm1)

        c = lax.axis_index("c")
        sid = lax.axis_index("s")
        row0 = sid * RP
        cnt = jnp.where(c == 0, NB0, NB1)
        blk0b = jnp.where(c == 0, sid * NB0, NS * NB0 + sid * NB1)
        iota16 = lax.iota(_i32, 16)

        pltpu.sync_copy(srcb_hbm.at[pl.ds(blk0b, NB0)], srcb)
        pltpu.sync_copy(dstb_hbm.at[pl.ds(blk0b, NB0)], dstb)

        pltpu.sync_copy(z16.at[pl.ds(row0, RP)], denom_s.at[pl.ds(row0, RP)])
        plsc.subcore_barrier()

        pltpu.async_copy(q_hbm.at[dstb.at[0, pl.ds(0, BA)]], Qbuf[0], sqb[0])
        pltpu.async_copy(k_hbm.at[srcb.at[0, pl.ds(0, BA)]], Kbuf[0], skb[0])

        def _pair_a(bp, carry):
            for hb in range(2):
                b = bp * 2 + hb
                pb = hb

                @pl.when(bp > 0)
                def _wait_prev():
                    pltpu.make_async_copy(
                        apb[pb], denom_s.at[dstb.at[0]], sscb[pb]).wait()
                    pltpu.make_async_copy(
                        apb[pb], a_hbm.at[pl.ds(0, BB)], sstb[pb]).wait()

                for hf in range(QPB):
                    gb = hf % 2
                    nb_ = (hf + 1) % 2
                    pltpu.make_async_copy(
                        q_hbm.at[dstb.at[0, pl.ds(0, BA)]], Qbuf[gb],
                        sqb[gb]).wait()
                    pltpu.make_async_copy(
                        k_hbm.at[srcb.at[0, pl.ds(0, BA)]], Kbuf[gb],
                        skb[gb]).wait()
                    if hf < QPB - 1:
                        nxb, nxo = b, (hf + 1) * BA
                    else:
                        nxb, nxo = jnp.minimum(b + 1, cnt - 1), 0
                    pltpu.async_copy(q_hbm.at[dstb.at[nxb, pl.ds(nxo, BA)]],
                                     Qbuf[nb_], sqb[nb_])
                    pltpu.async_copy(k_hbm.at[srcb.at[nxb, pl.ds(nxo, BA)]],
                                     Kbuf[nb_], skb[nb_])

                    Qg, Kg = Qbuf[gb], Kbuf[gb]

                    def _dotrow(e, ecarry, Qg=Qg, Kg=Kg, hf=hf):
                        vec = jnp.zeros((16,), _f32)
                        for h in range(heads - 1, -1, -1):
                            c0 = h * ch
                            p = jnp.zeros((16,), _f32)
                            for u in range(ch // 32):
                                qv = Qg[e, pl.ds(c0 + u * 32, 32)]
                                kv = Kg[e, pl.ds(c0 + u * 32, 32)]
                                qa, qb = plsc.unpack(
                                    qv, format=plsc.PackFormat.INTERLEAVED,
                                    preferred_element_type=_f32)
                                ka, kb = plsc.unpack(
                                    kv, format=plsc.PackFormat.INTERLEAVED,
                                    preferred_element_type=_f32)
                                p = p + qa * ka + qb * kb
                            vec = jnp.where(iota16 == h, jnp.sum(p), vec)
                        lgt[hf * BA + e, :] = vec
                        return ecarry
                    lax.fori_loop(0, BA, _dotrow, 0)

                eidb = (blk0b + b) * BB
                lmask = lax.iota(_i32, 16) < heads

                def _expb(r, ecarry):
                    ok = jnp.logical_and(lmask, (eidb + r) < E)
                    apb[pb][r, :] = jnp.where(ok, jnp.exp(lgt[r, :] * scale),
                                              0.0)
                    return ecarry
                lax.fori_loop(0, BB, _expb, 0, unroll=4)

                pltpu.async_copy(apb[pb], denom_s.at[dstb.at[b]], sscb[pb],
                                 add=True)
                pltpu.async_copy(apb[pb],
                                 a_hbm.at[pl.ds((blk0b + b) * BB, BB)],
                                 sstb[pb])
            return carry
        lax.fori_loop(0, cnt // 2, _pair_a, 0)

        pltpu.make_async_copy(q_hbm.at[dstb.at[0, pl.ds(0, BA)]], Qbuf[0],
                              sqb[0]).wait()
        pltpu.make_async_copy(k_hbm.at[srcb.at[0, pl.ds(0, BA)]], Kbuf[0],
                              skb[0]).wait()
        for pb in range(2):
            pltpu.make_async_copy(apb[pb], denom_s.at[dstb.at[0]],
                                  sscb[pb]).wait()
            pltpu.make_async_copy(apb[pb], a_hbm.at[pl.ds(0, BB)],
                                  sstb[pb]).wait()

        plsc.subcore_barrier()
        pltpu.sync_copy(denom_s.at[pl.ds(row0, RP)],
                        denom_hbm.at[c, pl.ds(row0, RP)])

        for g in range(ngrp):
            pltpu.sync_copy(z64.at[pl.ds(row0, RP)], out_s.at[pl.ds(row0, RP)])
            plsc.subcore_barrier()

            for pr in range(3):
                pltpu.async_copy(v_hbms[g].at[srcb.at[pr]], Vb[pr], svb[pr])
                pltpu.async_copy(a_hbm.at[pl.ds((blk0b + pr) * BB, BB)],
                                 avb[pr], sab[pr])

            def _quad_b(bp, carry, g=g):
                for qb in range(4):
                    b = bp * 4 + qb
                    u_ = qb
                    m_ = qb % 2

                    if qb < 2:
                        @pl.when(bp > 0)
                        def _wait_msg():
                            pltpu.make_async_copy(
                                mgb[m_], out_s.at[dstb.at[0]], smb[m_]).wait()
                    else:
                        pltpu.make_async_copy(
                            mgb[m_], out_s.at[dstb.at[0]], smb[m_]).wait()

                    pltpu.make_async_copy(v_hbms[g].at[srcb.at[0]], Vb[u_],
                                          svb[u_]).wait()
                    pltpu.make_async_copy(a_hbm.at[pl.ds(0, BB)], avb[u_],
                                          sab[u_]).wait()
                    nxb = jnp.minimum(b + 3, cnt - 1)
                    nu = (qb + 3) % 4
                    pltpu.async_copy(v_hbms[g].at[srcb.at[nxb]], Vb[nu],
                                     svb[nu])
                    pltpu.async_copy(a_hbm.at[pl.ds((blk0b + nxb) * BB, BB)],
                                     avb[nu], sab[nu])

                    Vg, Mg, Ag = Vb[u_], mgb[m_], avb[u_]

                    def _edge(e, ecarry, Vg=Vg, Mg=Mg, Ag=Ag, g=g):
                        arow = Ag[e, :]
                        for u in range(4):
                            h = (g * 64 + u * 16) // ch
                            Mg[e, pl.ds(u * 16, 16)] = (
                                Vg[e, pl.ds(u * 16, 16)] * arow[h])
                        return ecarry
                    lax.fori_loop(0, BB, _edge, 0, unroll=2)
                    pltpu.async_copy(Mg, out_s.at[dstb.at[b]], smb[m_],
                                     add=True)
                return carry
            lax.fori_loop(0, cnt // 4, _quad_b, 0)

            for pr in range(3):
                pltpu.make_async_copy(v_hbms[g].at[srcb.at[0]], Vb[pr],
                                      svb[pr]).wait()
                pltpu.make_async_copy(a_hbm.at[pl.ds(0, BB)], avb[pr],
                                      sab[pr]).wait()
            for m_ in range(2):
                pltpu.make_async_copy(mgb[m_], out_s.at[dstb.at[0]],
                                      smb[m_]).wait()

            plsc.subcore_barrier()
            pltpu.sync_copy(out_s.at[pl.ds(row0, RP)],
                            out_hbm.at[c, g, pl.ds(row0, RP)])

    return pl.kernel(
        body,
        compiler_params=pltpu.CompilerParams(use_tc_tiling_on_sc=False,
                                             needs_layout_passes=False),
        out_type=[
            jax.ShapeDtypeStruct((NC, NP, 16), _f32),
            jax.ShapeDtypeStruct((NC, ngrp, NP, 64), _f32),
            jax.ShapeDtypeStruct((EP, 16), _f32),
        ],
        mesh=mesh,
        scratch_types=[
            pltpu.VMEM((NB0, BB), _i32),
            pltpu.VMEM((NB0, BB), _i32),
            pltpu.VMEM((BA, D), jnp.bfloat16),
            pltpu.VMEM((BA, D), jnp.bfloat16),
            pltpu.VMEM((BA, D), jnp.bfloat16),
            pltpu.VMEM((BA, D), jnp.bfloat16),
            pltpu.VMEM((BB, 16), _f32),
            pltpu.VMEM((BB, 16), _f32),
            pltpu.VMEM((BB, 16), _f32),
            pltpu.VMEM((BB, 64), _f32),
            pltpu.VMEM((BB, 64), _f32),
            pltpu.VMEM((BB, 64), _f32),
            pltpu.VMEM((BB, 64), _f32),
            pltpu.VMEM((BB, 64), _f32),
            pltpu.VMEM((BB, 64), _f32),
            pltpu.VMEM((BB, 16), _f32),
            pltpu.VMEM((BB, 16), _f32),
            pltpu.VMEM((BB, 16), _f32),
            pltpu.VMEM((BB, 16), _f32),
            pltpu.VMEM_SHARED((NP, 16), _f32),
            pltpu.VMEM_SHARED((NP, 64), _f32),
        ] + [pltpu.SemaphoreType.DMA] * 18,
    )


def kernel(x, edge_index, Wq1, bq1, Wk1, bk1, Wv1, bv1, Ws1, bs1,
           Wq2, bq2, Wk2, bk2, Wv2, bv2, Ws2, bs2):
    Wcat1 = jnp.concatenate([Wq1, Wk1, Ws1, Wv1], axis=1)
    bcat1 = jnp.concatenate([bq1, bk1, bs1, bv1]).reshape(1, -1)
    Wcat2 = jnp.concatenate([Wq2, Wk2, Ws2, Wv2], axis=1)
    bcat2 = jnp.concatenate([bq2, bk2, bs2, bv2]).reshape(1, -1)

    pad = jnp.zeros((EP - E,), _i32)
    srcb2d = jnp.concatenate([edge_index[0], pad]).reshape(NW * NBB, BB)
    dstb2d = jnp.concatenate([edge_index[1], pad]).reshape(NW * NBB, BB)
    z16 = jnp.zeros((NP, 16), _f32)
    z64 = jnp.zeros((NP, 64), _f32)

    tc1_out = _tc1(x, Wcat1, bcat1)
    q1, k1, s1 = tc1_out[0], tc1_out[1], tc1_out[2]
    v1g = tc1_out[3:]
    sc1 = _make_sc_layer(D_MID, HEADS, 8, 1.0 / 8.0)
    denp1, outp1, _ = sc1(q1, k1, *v1g, srcb2d, dstb2d, z16, z64)

    tc2_out = _tc2(outp1, denp1, s1, Wcat2, bcat2)
    q2, k2, s2 = tc2_out[0], tc2_out[1], tc2_out[2]
    v2g = tc2_out[3:]
    sc2 = _make_sc_layer(D_OUT, 1, 4, 1.0 / 16.0)
    denp2, outp2, _ = sc2(q2, k2, *v2g, srcb2d, dstb2d, z16, z64)

    return _tc3(outp2, denp2, s2)
